# Initial kernel scaffold; baseline (speedup 1.0000x reference)
import jax
import jax.numpy as jnp
from jax.experimental import pallas as pl


def kernel(x, edge_index, batch, W1, b1, g1, be1, W2, b2, g2, be2, W3, b3, g3, be3, M1w, M1b, M2w, M2b, M3w, M3b, M4w, M4b):
    raise NotImplementedError("write your pallas kernel here")


# speedup vs baseline: 10.8393x; 10.8393x over previous
import functools
import math

import jax
import jax.numpy as jnp
from jax import lax
from jax.experimental import pallas as pl
from jax.experimental.pallas import tpu as pltpu
import jax.experimental.pallas.tpu_sc as plsc

_N = 10000
_E = 320000
_D = 128
_G = 64
_NC, _NS = 2, 16
_NW = _NC * _NS
_K = 80
_EPW = _E // _NW
_NCH = _EPW // _K
_RPT = _N // _NS
_ZB = 1000
_SB = 200
_BNS = 1.0 / math.sqrt(1.0 + 1e-5)

_BM = 2000
_NB = _N // _BM

_sc_mesh = plsc.VectorSubcoreMesh(
    core_axis_name="c", subcore_axis_name="s", num_cores=_NC, num_subcores=_NS
)


def _fill1d(ref, n, val):
    def _st(i, carry):
        ref[pl.ds(i * 16, 16)] = jnp.full((16,), val, jnp.float32)
        return carry

    lax.fori_loop(0, n // 16, _st, 0)
    if n % 16:
        ref[pl.ds(n - 16, 16)] = jnp.full((16,), val, jnp.float32)


def _deg_body(dst_hbm, degp_hbm, dacc, didx, ones_v, dbuf):
    c = lax.axis_index("c")
    s = lax.axis_index("s")
    wid = c * _NS + s

    _fill1d(ones_v, _K, 1.0)
    _fill1d(dbuf, _ZB, 0.0)

    @pl.when(s < _N // _ZB)
    def _():
        pltpu.sync_copy(dbuf, dacc.at[pl.ds(s * _ZB, _ZB)])

    plsc.subcore_barrier()

    base = wid * _EPW

    def _step(i, carry):
        pltpu.sync_copy(dst_hbm.at[pl.ds(base + i * _K, _K)], didx)
        pltpu.sync_copy(ones_v, dacc.at[didx], add=True)
        return carry

    lax.fori_loop(0, _NCH, _step, 0)
    plsc.subcore_barrier()

    @pl.when(s < _N // _ZB)
    def _():
        pltpu.sync_copy(dacc.at[pl.ds(s * _ZB, _ZB)], dbuf)
        pltpu.sync_copy(dbuf, degp_hbm.at[pl.ds(c * _N + s * _ZB, _ZB)])


_deg_kernel = functools.partial(
    pl.kernel,
    out_type=jax.ShapeDtypeStruct((_NC * _N,), jnp.float32),
    mesh=_sc_mesh,
    scratch_types=[
        pltpu.VMEM_SHARED((_N,), jnp.float32),
        pltpu.VMEM((_K,), jnp.int32),
        pltpu.VMEM((_K,), jnp.float32),
        pltpu.VMEM((_ZB,), jnp.float32),
    ],
)(_deg_body)


def _spmm_body(u_hbm, src_hbm, dst_hbm, sp_hbm, sacc, sidx, didx, rows, stage,
               sem):
    c = lax.axis_index("c")
    s = lax.axis_index("s")
    wid = c * _NS + s

    @pl.when(s < _N // _ZB)
    def _():
        for j in range(_ZB // _SB):
            r0 = s * _ZB + j * _SB
            pltpu.sync_copy(u_hbm.at[pl.ds(r0, _SB)], stage)
            pltpu.sync_copy(stage, sacc.at[pl.ds(r0, _SB)])

    plsc.subcore_barrier()

    base = wid * _EPW

    def _step(i, carry):
        off = base + i * _K
        pltpu.sync_copy(src_hbm.at[pl.ds(off, _K)], sidx)
        pltpu.sync_copy(dst_hbm.at[pl.ds(off, _K)], didx)
        pltpu.async_copy(u_hbm.at[sidx], rows, sem).wait()
        pltpu.sync_copy(rows, sacc.at[didx], add=True)
        return carry

    lax.fori_loop(0, _NCH, _step, 0)
    plsc.subcore_barrier()

    @pl.when(s < _N // _ZB)
    def _():
        for j in range(_ZB // _SB):
            r0 = s * _ZB + j * _SB
            pltpu.sync_copy(sacc.at[pl.ds(r0, _SB)], stage)
            pltpu.sync_copy(stage, sp_hbm.at[c, pl.ds(r0, _SB)])


_spmm_kernel = functools.partial(
    pl.kernel,
    out_type=jax.ShapeDtypeStruct((_NC, _N, _D), jnp.float32),
    mesh=_sc_mesh,
    scratch_types=[
        pltpu.VMEM_SHARED((_N, _D), jnp.float32),
        pltpu.VMEM((_K,), jnp.int32),
        pltpu.VMEM((_K,), jnp.int32),
        pltpu.VMEM((_K, _D), jnp.float32),
        pltpu.VMEM((_SB, _D), jnp.float32),
        pltpu.SemaphoreType.DMA,
    ],
)(_spmm_body)


def _mm1_body(x_ref, w_ref, d0_ref, d1_ref, u_ref, dinv_ref):
    deg = d0_ref[0] + d1_ref[0] + 1.0
    dinv = lax.rsqrt(deg)
    dinv_ref[0] = dinv
    u_ref[...] = jnp.dot(x_ref[...] * dinv, w_ref[...],
                         preferred_element_type=jnp.float32)


def _mm1(x, W1, d0, d1):
    return pl.pallas_call(
        _mm1_body,
        grid=(_NB,),
        in_specs=[
            pl.BlockSpec((_BM, _D), lambda i: (i, 0)),
            pl.BlockSpec((_D, _D), lambda i: (0, 0)),
            pl.BlockSpec((1, _BM, 1), lambda i: (i, 0, 0)),
            pl.BlockSpec((1, _BM, 1), lambda i: (i, 0, 0)),
        ],
        out_specs=[
            pl.BlockSpec((_BM, _D), lambda i: (i, 0)),
            pl.BlockSpec((1, _BM, 1), lambda i: (i, 0, 0)),
        ],
        out_shape=[
            jax.ShapeDtypeStruct((_N, _D), jnp.float32),
            jax.ShapeDtypeStruct((_NB, _BM, 1), jnp.float32),
        ],
    )(x, W1, d0, d1)


def _mid_body(s0_ref, s1_ref, u_ref, dinv_ref, b_ref, g_ref, be_ref, w_ref,
              out_ref):
    dinv = dinv_ref[0]
    t = (s0_ref[...] + s1_ref[...] - u_ref[...]) * dinv + b_ref[...]
    h = jnp.maximum(t * (g_ref[...] * _BNS) + be_ref[...], 0.0)
    out_ref[...] = jnp.dot(h * dinv, w_ref[...],
                           preferred_element_type=jnp.float32)


def _mid(s0, s1, u, dinv, b, g, be, W):
    return pl.pallas_call(
        _mid_body,
        grid=(_NB,),
        in_specs=[
            pl.BlockSpec((_BM, _D), lambda i: (i, 0)),
            pl.BlockSpec((_BM, _D), lambda i: (i, 0)),
            pl.BlockSpec((_BM, _D), lambda i: (i, 0)),
            pl.BlockSpec((1, _BM, 1), lambda i: (i, 0, 0)),
            pl.BlockSpec((1, _D), lambda i: (0, 0)),
            pl.BlockSpec((1, _D), lambda i: (0, 0)),
            pl.BlockSpec((1, _D), lambda i: (0, 0)),
            pl.BlockSpec((_D, _D), lambda i: (0, 0)),
        ],
        out_specs=pl.BlockSpec((_BM, _D), lambda i: (i, 0)),
        out_shape=jax.ShapeDtypeStruct((_N, _D), jnp.float32),
    )(s0, s1, u, dinv, b, g, be, W)


def _head_body(s0_ref, s1_ref, u_ref, dinv_ref, b_ref, g_ref, be_ref,
               batch_ref, m1w, m1b, m2w, m2b, m3w, m3b, m4w, m4b,
               out_ref, pacc, cacc):
    i = pl.program_id(0)

    @pl.when(i == 0)
    def _():
        pacc[...] = jnp.zeros_like(pacc)
        cacc[...] = jnp.zeros_like(cacc)

    dinv = dinv_ref[0]
    t = (s0_ref[...] + s1_ref[...] - u_ref[...]) * dinv + b_ref[...]
    h = jnp.maximum(t * (g_ref[...] * _BNS) + be_ref[...], 0.0)

    bb = batch_ref[0]
    gids = lax.broadcasted_iota(jnp.int32, (_G, 1), 0)
    oh = jnp.where(bb == gids, 1.0, 0.0)
    pacc[...] += jnp.dot(oh, h, preferred_element_type=jnp.float32)
    cacc[...] += jnp.sum(oh, axis=1, keepdims=True)

    @pl.when(i == _NB - 1)
    def _():
        pooled = pacc[...] / jnp.maximum(cacc[...], 1.0)
        z = jnp.maximum(jnp.dot(pooled, m1w[...]) + m1b[...], 0.0)
        z = jnp.maximum(jnp.dot(z, m2w[...]) + m2b[...], 0.0)
        z = jnp.maximum(jnp.dot(z, m3w[...]) + m3b[...], 0.0)
        out_ref[...] = jnp.dot(z, m4w[...]) + m4b[...]


def _head(s0, s1, u, dinv, b, g, be, batchR, M1w, M1b, M2w, M2b, M3w, M3b,
          M4w, M4b):
    wspec = lambda: pl.BlockSpec(None, lambda i: (0, 0))
    return pl.pallas_call(
        _head_body,
        grid=(_NB,),
        in_specs=[
            pl.BlockSpec((_BM, _D), lambda i: (i, 0)),
            pl.BlockSpec((_BM, _D), lambda i: (i, 0)),
            pl.BlockSpec((_BM, _D), lambda i: (i, 0)),
            pl.BlockSpec((1, _BM, 1), lambda i: (i, 0, 0)),
            pl.BlockSpec((1, _D), lambda i: (0, 0)),
            pl.BlockSpec((1, _D), lambda i: (0, 0)),
            pl.BlockSpec((1, _D), lambda i: (0, 0)),
            pl.BlockSpec((1, 1, _BM), lambda i: (i, 0, 0)),
            pl.BlockSpec((_D, _D), lambda i: (0, 0)),
            pl.BlockSpec((1, _D), lambda i: (0, 0)),
            pl.BlockSpec((_D, _G), lambda i: (0, 0)),
            pl.BlockSpec((1, _G), lambda i: (0, 0)),
            pl.BlockSpec((_G, 32), lambda i: (0, 0)),
            pl.BlockSpec((1, 32), lambda i: (0, 0)),
            pl.BlockSpec((32, 2), lambda i: (0, 0)),
            pl.BlockSpec((1, 2), lambda i: (0, 0)),
        ],
        out_specs=pl.BlockSpec((_G, 2), lambda i: (0, 0)),
        out_shape=jax.ShapeDtypeStruct((_G, 2), jnp.float32),
        scratch_shapes=[
            pltpu.VMEM((_G, _D), jnp.float32),
            pltpu.VMEM((_G, 1), jnp.float32),
        ],
    )(s0, s1, u, dinv, b, g, be, batchR, M1w, M1b, M2w, M2b, M3w, M3b,
      M4w, M4b)


def kernel(x, edge_index, batch, W1, b1, g1, be1, W2, b2, g2, be2,
           W3, b3, g3, be3, M1w, M1b, M2w, M2b, M3w, M3b, M4w, M4b):
    src = edge_index[0].astype(jnp.int32)
    dst = edge_index[1].astype(jnp.int32)


    degp = _deg_kernel(dst)
    d0 = degp[:_N].reshape(_NB, _BM, 1)
    d1 = degp[_N:].reshape(_NB, _BM, 1)

    u1, dinvR = _mm1(x, W1, d0, d1)

    sp1 = _spmm_kernel(u1, src, dst)
    u2 = _mid(sp1[0], sp1[1], u1, dinvR, b1.reshape(1, _D),
              g1.reshape(1, _D), be1.reshape(1, _D), W2)

    sp2 = _spmm_kernel(u2, src, dst)
    u3 = _mid(sp2[0], sp2[1], u2, dinvR, b2.reshape(1, _D),
              g2.reshape(1, _D), be2.reshape(1, _D), W3)

    sp3 = _spmm_kernel(u3, src, dst)
    out = _head(sp3[0], sp3[1], u3, dinvR, b3.reshape(1, _D),
                g3.reshape(1, _D), be3.reshape(1, _D),
                batch.astype(jnp.int32).reshape(_NB, 1, _BM),
                M1w, M1b.reshape(1, _D), M2w, M2b.reshape(1, _G),
                M3w, M3b.reshape(1, 32), M4w, M4b.reshape(1, 2))
    return out
